# Initial kernel scaffold; baseline (speedup 1.0000x reference)
import jax
import jax.numpy as jnp
from jax.experimental import pallas as pl


def kernel(x, edge_index, edge_attr, batch, W1, b1, W3, b3, W5, b5, W7, b7):
    raise NotImplementedError("write your pallas kernel here")


# speedup vs baseline: 5.9249x; 5.9249x over previous
import functools

import jax
import jax.numpy as jnp
from jax import lax
from jax.experimental import pallas as pl
from jax.experimental.pallas import tpu as pltpu
from jax.experimental.pallas import tpu_sc as plsc

NG = 32
F = 64
NPAD = 51200
EP = 819200
NWORK = 32
EPW = EP // NWORK
ACH = 6400
CCH = 8192
BLK = 6400
NBLK = NPAD // BLK


def _worker_id():
    return lax.axis_index("s") * 2 + lax.axis_index("c")


def _zero_f32(ref, n):
    z = jnp.zeros((16,), jnp.float32)

    def body(i, _):
        ref[pl.ds(i * 16, 16)] = z
        return 0

    lax.fori_loop(0, n // 16, body, 0, unroll=8)


def _sc_deg_pack_body(src_hbm, dst_hbm, w_hbm, degp_hbm, pk_hbm,
                      src_v, dst_v, w_v, pk_v, acc_v):
    wid = _worker_id()
    _zero_f32(acc_v, NPAD)
    base = wid * EPW
    for ci in range(EPW // ACH):
        off = pl.multiple_of(base + ci * ACH, 8)
        pltpu.sync_copy(src_hbm.at[pl.ds(off, ACH)], src_v)
        pltpu.sync_copy(dst_hbm.at[pl.ds(off, ACH)], dst_v)
        pltpu.sync_copy(w_hbm.at[pl.ds(off, ACH)], w_v)

        def body(i, _):
            sl = pl.ds(i * 16, 16)
            sv = src_v[sl]
            dv = dst_v[sl]
            plsc.addupdate_scatter(acc_v, [dv], w_v[sl])
            pk_v[sl] = jnp.bitwise_or(sv, lax.shift_left(dv, 16))
            return 0

        lax.fori_loop(0, ACH // 16, body, 0, unroll=4)
        pltpu.sync_copy(pk_v, pk_hbm.at[pl.ds(off, ACH)])
    pltpu.sync_copy(acc_v, degp_hbm.at[wid])


def _sc_deg_pack(src, dst, w):
    return pl.kernel(
        _sc_deg_pack_body,
        out_type=(
            jax.ShapeDtypeStruct((NWORK, NPAD), jnp.float32),
            jax.ShapeDtypeStruct((EP,), jnp.int32),
        ),
        mesh=plsc.VectorSubcoreMesh(core_axis_name="c", subcore_axis_name="s"),
        scratch_types=(
            pltpu.VMEM((ACH,), jnp.int32),
            pltpu.VMEM((ACH,), jnp.int32),
            pltpu.VMEM((ACH,), jnp.float32),
            pltpu.VMEM((ACH,), jnp.int32),
            pltpu.VMEM((NPAD,), jnp.float32),
        ),
        compiler_params=pltpu.CompilerParams(needs_layout_passes=False),
    )(src, dst, w)


def _sc_msg_body(pk_hbm, w_hbm, xsT_hbm, accT_hbm, pk_v, w_v, xs_v, acc_v):
    wid = _worker_id()
    for fi in range(2):
        f = wid * 2 + fi
        pltpu.sync_copy(xsT_hbm.at[f], xs_v)
        _zero_f32(acc_v, NPAD)

        def chunk(ci, _):
            off = pl.multiple_of(ci * CCH, 8)
            pltpu.sync_copy(pk_hbm.at[pl.ds(off, CCH)], pk_v)
            pltpu.sync_copy(w_hbm.at[pl.ds(off, CCH)], w_v)

            def body(i, _):
                sl = pl.ds(i * 16, 16)
                pk = pk_v[sl]
                sv = jnp.bitwise_and(pk, 0xFFFF)
                dv = lax.shift_right_logical(pk, 16)
                g = plsc.load_gather(xs_v, [sv])
                plsc.addupdate_scatter(acc_v, [dv], g * w_v[sl])
                return 0

            lax.fori_loop(0, CCH // 16, body, 0, unroll=4)
            return 0

        lax.fori_loop(0, EP // CCH, chunk, 0)
        pltpu.sync_copy(acc_v, accT_hbm.at[f])


def _sc_msg(pk, w, xsT):
    return pl.kernel(
        _sc_msg_body,
        out_type=jax.ShapeDtypeStruct((F, NPAD), jnp.float32),
        mesh=plsc.VectorSubcoreMesh(core_axis_name="c", subcore_axis_name="s"),
        scratch_types=(
            pltpu.VMEM((CCH,), jnp.int32),
            pltpu.VMEM((CCH,), jnp.float32),
            pltpu.VMEM((NPAD,), jnp.float32),
            pltpu.VMEM((NPAD,), jnp.float32),
        ),
        compiler_params=pltpu.CompilerParams(needs_layout_passes=False),
    )(pk, w, xsT)


def _tc_first_body(x_ref, degp_ref, w1_ref, xsT_ref, dis_ref):
    deg = jnp.sum(degp_ref[...], axis=0, keepdims=True) + 1.0
    dis = jnp.where(deg > 0, lax.rsqrt(deg), 0.0)
    xwT = lax.dot_general(w1_ref[...], x_ref[...], (((0,), (1,)), ((), ())))
    xsT_ref[...] = xwT * dis
    dis_ref[...] = dis


def _tc_first(x_p, degp, W1):
    cin = W1.shape[0]
    return pl.pallas_call(
        _tc_first_body,
        grid=(NBLK,),
        in_specs=[
            pl.BlockSpec((BLK, cin), lambda i: (i, 0)),
            pl.BlockSpec((NWORK, BLK), lambda i: (0, i)),
            pl.BlockSpec((cin, F), lambda i: (0, 0)),
        ],
        out_specs=[
            pl.BlockSpec((F, BLK), lambda i: (0, i)),
            pl.BlockSpec((1, BLK), lambda i: (0, i)),
        ],
        out_shape=[
            jax.ShapeDtypeStruct((F, NPAD), jnp.float32),
            jax.ShapeDtypeStruct((1, NPAD), jnp.float32),
        ],
    )(x_p, degp, W1)


def _tc_layer_body(accT_ref, xsT_ref, dis_ref, w_ref, bT_ref, out_ref):
    dis = dis_ref[...]
    hT = (accT_ref[...] + xsT_ref[...]) * dis + bT_ref[...]
    hT = jnp.maximum(hT, 0.0)
    xwT = lax.dot_general(w_ref[...], hT, (((0,), (0,)), ((), ())))
    out_ref[...] = xwT * dis


def _tc_layer(accT, xsT, dis, W, bT):
    return pl.pallas_call(
        _tc_layer_body,
        grid=(NBLK,),
        in_specs=[
            pl.BlockSpec((F, BLK), lambda i: (0, i)),
            pl.BlockSpec((F, BLK), lambda i: (0, i)),
            pl.BlockSpec((1, BLK), lambda i: (0, i)),
            pl.BlockSpec((F, F), lambda i: (0, 0)),
            pl.BlockSpec((F, 1), lambda i: (0, 0)),
        ],
        out_specs=pl.BlockSpec((F, BLK), lambda i: (0, i)),
        out_shape=jax.ShapeDtypeStruct((F, NPAD), jnp.float32),
    )(accT, xsT, dis, W, bT)


def _tc_pool_body(accT_ref, xsT_ref, dis_ref, bT_ref, batch_ref, w7_ref,
                  b7_ref, out_ref, sums_s, counts_s):
    i = pl.program_id(0)

    @pl.when(i == 0)
    def _():
        sums_s[...] = jnp.zeros_like(sums_s)
        counts_s[...] = jnp.zeros_like(counts_s)

    hT = (accT_ref[...] + xsT_ref[...]) * dis_ref[...] + bT_ref[...]
    gids = lax.broadcasted_iota(jnp.int32, (NG, BLK), 0)
    m = (batch_ref[...] == gids).astype(jnp.float32)
    sums_s[...] += lax.dot_general(m, hT, (((1,), (1,)), ((), ())))
    counts_s[...] += jnp.broadcast_to(
        jnp.sum(m, axis=1, keepdims=True), (NG, 128))

    @pl.when(i == pl.num_programs(0) - 1)
    def _():
        g = sums_s[...] / jnp.maximum(counts_s[:, 0:1], 1.0)
        out_ref[...] = jnp.dot(g, w7_ref[...]) + b7_ref[...]


def _tc_pool(accT, xsT, dis, bT, batch_p, W7, b7_2d):
    return pl.pallas_call(
        _tc_pool_body,
        grid=(NBLK,),
        in_specs=[
            pl.BlockSpec((F, BLK), lambda i: (0, i)),
            pl.BlockSpec((F, BLK), lambda i: (0, i)),
            pl.BlockSpec((1, BLK), lambda i: (0, i)),
            pl.BlockSpec((F, 1), lambda i: (0, 0)),
            pl.BlockSpec((1, BLK), lambda i: (0, i)),
            pl.BlockSpec((F, 2), lambda i: (0, 0)),
            pl.BlockSpec((1, 2), lambda i: (0, 0)),
        ],
        out_specs=pl.BlockSpec((NG, 2), lambda i: (0, 0)),
        out_shape=jax.ShapeDtypeStruct((NG, 2), jnp.float32),
        scratch_shapes=[
            pltpu.VMEM((NG, F), jnp.float32),
            pltpu.VMEM((NG, 128), jnp.float32),
        ],
    )(accT, xsT, dis, bT, batch_p, W7, b7_2d)


def kernel(x, edge_index, edge_attr, batch, W1, b1, W3, b3, W5, b5, W7, b7):
    N = x.shape[0]
    E = edge_index.shape[1]
    src = edge_index[0].astype(jnp.int32)
    dst = edge_index[1].astype(jnp.int32)
    pad_i = jnp.full((EP - E,), N, jnp.int32)
    src_p = jnp.concatenate([src, pad_i])
    dst_p = jnp.concatenate([dst, pad_i])
    w_p = jnp.concatenate([edge_attr.astype(jnp.float32),
                           jnp.zeros((EP - E,), jnp.float32)])
    x_p = jnp.pad(x.astype(jnp.float32), ((0, NPAD - N), (0, 0)))
    batch_p = jnp.concatenate(
        [batch.astype(jnp.int32), jnp.full((NPAD - N,), NG, jnp.int32)]
    ).reshape(1, NPAD)

    degp, pk = _sc_deg_pack(src_p, dst_p, w_p)
    xs1T, dis = _tc_first(x_p, degp, W1)
    acc1T = _sc_msg(pk, w_p, xs1T)
    xs2T = _tc_layer(acc1T, xs1T, dis, W3, b1.reshape(F, 1))
    acc2T = _sc_msg(pk, w_p, xs2T)
    xs3T = _tc_layer(acc2T, xs2T, dis, W5, b3.reshape(F, 1))
    acc3T = _sc_msg(pk, w_p, xs3T)
    return _tc_pool(acc3T, xs3T, dis, b5.reshape(F, 1), batch_p, W7,
                    b7.reshape(1, 2))


# speedup vs baseline: 17.4279x; 2.9415x over previous
import functools

import jax
import jax.numpy as jnp
from jax import lax
from jax.experimental import pallas as pl
from jax.experimental.pallas import tpu as pltpu
from jax.experimental.pallas import tpu_sc as plsc

NG = 32
F = 64
NPAD = 51200
EP = 819200
NWORK = 32
EPW = EP // NWORK
ACH = 6400
CCH = 6400
BLK = 6400
NBLK = NPAD // BLK


def _worker_id():
    return lax.axis_index("s") * 2 + lax.axis_index("c")


def _zero_f32(ref, n):
    z = jnp.zeros((16,), jnp.float32)

    def body(i, _):
        ref[pl.ds(i * 16, 16)] = z
        return 0

    lax.fori_loop(0, n // 16, body, 0, unroll=8)


def _sc_deg_pack_body(src_hbm, dst_hbm, w_hbm, degp_hbm, pk_hbm,
                      src_v, dst_v, w_v, pk_v, acc_v):
    wid = _worker_id()
    _zero_f32(acc_v, NPAD)
    base = wid * EPW
    for ci in range(EPW // ACH):
        off = pl.multiple_of(base + ci * ACH, 8)
        pltpu.sync_copy(src_hbm.at[pl.ds(off, ACH)], src_v)
        pltpu.sync_copy(dst_hbm.at[pl.ds(off, ACH)], dst_v)
        pltpu.sync_copy(w_hbm.at[pl.ds(off, ACH)], w_v)

        def body(i, _):
            sl = pl.ds(i * 16, 16)
            sv = src_v[sl]
            dv = dst_v[sl]
            plsc.addupdate_scatter(acc_v, [dv], w_v[sl])
            pk_v[sl] = jnp.bitwise_or(sv, lax.shift_left(dv, 16))
            return 0

        lax.fori_loop(0, ACH // 16, body, 0, unroll=4)
        pltpu.sync_copy(pk_v, pk_hbm.at[pl.ds(off, ACH)])
    pltpu.sync_copy(acc_v, degp_hbm.at[wid])


def _sc_deg_pack(src, dst, w):
    return pl.kernel(
        _sc_deg_pack_body,
        out_type=(
            jax.ShapeDtypeStruct((NWORK, NPAD), jnp.float32),
            jax.ShapeDtypeStruct((EP,), jnp.int32),
        ),
        mesh=plsc.VectorSubcoreMesh(core_axis_name="c", subcore_axis_name="s"),
        scratch_types=(
            pltpu.VMEM((ACH,), jnp.int32),
            pltpu.VMEM((ACH,), jnp.int32),
            pltpu.VMEM((ACH,), jnp.float32),
            pltpu.VMEM((ACH,), jnp.int32),
            pltpu.VMEM((NPAD,), jnp.float32),
        ),
        compiler_params=pltpu.CompilerParams(needs_layout_passes=False),
    )(src, dst, w)


def _sc_msg_body(pk_hbm, w_hbm, xsT_hbm, accT_hbm, pk_v, w_v, xs_v, acc_v,
                 sem0, sem1):
    wid = _worker_id()
    nch = EP // CCH
    sems = (sem0, sem1)

    def start(ci, b):
        off = pl.multiple_of(ci * CCH, 8)
        pltpu.make_async_copy(pk_hbm.at[pl.ds(off, CCH)], pk_v.at[b],
                              sems[b]).start()
        pltpu.make_async_copy(w_hbm.at[pl.ds(off, CCH)], w_v.at[b],
                              sems[b]).start()

    def wait(b):
        pltpu.make_async_copy(pk_hbm.at[pl.ds(0, CCH)], pk_v.at[b],
                              sems[b]).wait()
        pltpu.make_async_copy(w_hbm.at[pl.ds(0, CCH)], w_v.at[b],
                              sems[b]).wait()

    for fi in range(2):
        f = wid * 2 + fi
        start(0, 0)
        start(1, 1)
        pltpu.sync_copy(xsT_hbm.at[f], xs_v)
        _zero_f32(acc_v, NPAD)

        def outer(g, _):
            for b in range(2):
                ci = 2 * g + b
                wait(b)

                @plsc.parallel_loop(0, CCH, step=16, unroll=8)
                def body(i):
                    sl = pl.ds(i, 16)
                    pk = pk_v[b, sl]
                    sv = jnp.bitwise_and(pk, 0xFFFF)
                    dv = lax.shift_right_logical(pk, 16)
                    g16 = plsc.load_gather(xs_v, [sv])
                    plsc.addupdate_scatter(acc_v, [dv], g16 * w_v[b, sl])

                @pl.when(ci + 2 < nch)
                def _():
                    start(ci + 2, b)
            return 0

        lax.fori_loop(0, nch // 2, outer, 0)
        pltpu.sync_copy(acc_v, accT_hbm.at[f])


def _sc_msg(pk, w, xsT):
    return pl.kernel(
        _sc_msg_body,
        out_type=jax.ShapeDtypeStruct((F, NPAD), jnp.float32),
        mesh=plsc.VectorSubcoreMesh(core_axis_name="c", subcore_axis_name="s"),
        scratch_types=(
            pltpu.VMEM((2, CCH), jnp.int32),
            pltpu.VMEM((2, CCH), jnp.float32),
            pltpu.VMEM((NPAD,), jnp.float32),
            pltpu.VMEM((NPAD,), jnp.float32),
            pltpu.SemaphoreType.DMA,
            pltpu.SemaphoreType.DMA,
        ),
        compiler_params=pltpu.CompilerParams(needs_layout_passes=False),
    )(pk, w, xsT)


def _tc_first_body(x_ref, degp_ref, w1_ref, xsT_ref, dis_ref):
    deg = jnp.sum(degp_ref[...], axis=0, keepdims=True) + 1.0
    dis = jnp.where(deg > 0, lax.rsqrt(deg), 0.0)
    xwT = lax.dot_general(w1_ref[...], x_ref[...], (((0,), (1,)), ((), ())))
    xsT_ref[...] = xwT * dis
    dis_ref[...] = dis


def _tc_first(x_p, degp, W1):
    cin = W1.shape[0]
    return pl.pallas_call(
        _tc_first_body,
        grid=(NBLK,),
        in_specs=[
            pl.BlockSpec((BLK, cin), lambda i: (i, 0)),
            pl.BlockSpec((NWORK, BLK), lambda i: (0, i)),
            pl.BlockSpec((cin, F), lambda i: (0, 0)),
        ],
        out_specs=[
            pl.BlockSpec((F, BLK), lambda i: (0, i)),
            pl.BlockSpec((1, BLK), lambda i: (0, i)),
        ],
        out_shape=[
            jax.ShapeDtypeStruct((F, NPAD), jnp.float32),
            jax.ShapeDtypeStruct((1, NPAD), jnp.float32),
        ],
    )(x_p, degp, W1)


def _tc_layer_body(accT_ref, xsT_ref, dis_ref, w_ref, bT_ref, out_ref):
    dis = dis_ref[...]
    hT = (accT_ref[...] + xsT_ref[...]) * dis + bT_ref[...]
    hT = jnp.maximum(hT, 0.0)
    xwT = lax.dot_general(w_ref[...], hT, (((0,), (0,)), ((), ())))
    out_ref[...] = xwT * dis


def _tc_layer(accT, xsT, dis, W, bT):
    return pl.pallas_call(
        _tc_layer_body,
        grid=(NBLK,),
        in_specs=[
            pl.BlockSpec((F, BLK), lambda i: (0, i)),
            pl.BlockSpec((F, BLK), lambda i: (0, i)),
            pl.BlockSpec((1, BLK), lambda i: (0, i)),
            pl.BlockSpec((F, F), lambda i: (0, 0)),
            pl.BlockSpec((F, 1), lambda i: (0, 0)),
        ],
        out_specs=pl.BlockSpec((F, BLK), lambda i: (0, i)),
        out_shape=jax.ShapeDtypeStruct((F, NPAD), jnp.float32),
    )(accT, xsT, dis, W, bT)


def _tc_pool_body(accT_ref, xsT_ref, dis_ref, bT_ref, batch_ref, w7_ref,
                  b7_ref, out_ref, sums_s, counts_s):
    i = pl.program_id(0)

    @pl.when(i == 0)
    def _():
        sums_s[...] = jnp.zeros_like(sums_s)
        counts_s[...] = jnp.zeros_like(counts_s)

    hT = (accT_ref[...] + xsT_ref[...]) * dis_ref[...] + bT_ref[...]
    gids = lax.broadcasted_iota(jnp.int32, (NG, BLK), 0)
    m = (batch_ref[...] == gids).astype(jnp.float32)
    sums_s[...] += lax.dot_general(m, hT, (((1,), (1,)), ((), ())))
    counts_s[...] += jnp.broadcast_to(
        jnp.sum(m, axis=1, keepdims=True), (NG, 128))

    @pl.when(i == pl.num_programs(0) - 1)
    def _():
        g = sums_s[...] / jnp.maximum(counts_s[:, 0:1], 1.0)
        out_ref[...] = jnp.dot(g, w7_ref[...]) + b7_ref[...]


def _tc_pool(accT, xsT, dis, bT, batch_p, W7, b7_2d):
    return pl.pallas_call(
        _tc_pool_body,
        grid=(NBLK,),
        in_specs=[
            pl.BlockSpec((F, BLK), lambda i: (0, i)),
            pl.BlockSpec((F, BLK), lambda i: (0, i)),
            pl.BlockSpec((1, BLK), lambda i: (0, i)),
            pl.BlockSpec((F, 1), lambda i: (0, 0)),
            pl.BlockSpec((1, BLK), lambda i: (0, i)),
            pl.BlockSpec((F, 2), lambda i: (0, 0)),
            pl.BlockSpec((1, 2), lambda i: (0, 0)),
        ],
        out_specs=pl.BlockSpec((NG, 2), lambda i: (0, 0)),
        out_shape=jax.ShapeDtypeStruct((NG, 2), jnp.float32),
        scratch_shapes=[
            pltpu.VMEM((NG, F), jnp.float32),
            pltpu.VMEM((NG, 128), jnp.float32),
        ],
    )(accT, xsT, dis, bT, batch_p, W7, b7_2d)


def kernel(x, edge_index, edge_attr, batch, W1, b1, W3, b3, W5, b5, W7, b7):
    N = x.shape[0]
    E = edge_index.shape[1]
    src = edge_index[0].astype(jnp.int32)
    dst = edge_index[1].astype(jnp.int32)
    pad_i = jnp.full((EP - E,), N, jnp.int32)
    src_p = jnp.concatenate([src, pad_i])
    dst_p = jnp.concatenate([dst, pad_i])
    w_p = jnp.concatenate([edge_attr.astype(jnp.float32),
                           jnp.zeros((EP - E,), jnp.float32)])
    x_p = jnp.pad(x.astype(jnp.float32), ((0, NPAD - N), (0, 0)))
    batch_p = jnp.concatenate(
        [batch.astype(jnp.int32), jnp.full((NPAD - N,), NG, jnp.int32)]
    ).reshape(1, NPAD)

    degp, pk = _sc_deg_pack(src_p, dst_p, w_p)
    xs1T, dis = _tc_first(x_p, degp, W1)
    acc1T = _sc_msg(pk, w_p, xs1T)
    xs2T = _tc_layer(acc1T, xs1T, dis, W3, b1.reshape(F, 1))
    acc2T = _sc_msg(pk, w_p, xs2T)
    xs3T = _tc_layer(acc2T, xs2T, dis, W5, b3.reshape(F, 1))
    acc3T = _sc_msg(pk, w_p, xs3T)
    return _tc_pool(acc3T, xs3T, dis, b5.reshape(F, 1), batch_p, W7,
                    b7.reshape(1, 2))


# speedup vs baseline: 21.0794x; 1.2095x over previous
import functools

import jax
import jax.numpy as jnp
from jax import lax
from jax.experimental import pallas as pl
from jax.experimental.pallas import tpu as pltpu
from jax.experimental.pallas import tpu_sc as plsc

NG = 32
F = 64
NPAD = 51200
EP = 819200
NWORK = 32
EPW = EP // NWORK
ACH = 6400
CCH = 8192
BLK = 6400
NBLK = NPAD // BLK


def _worker_id():
    return lax.axis_index("s") * 2 + lax.axis_index("c")


def _zero_f32(ref, n):
    z = jnp.zeros((16,), jnp.float32)

    def body(i, _):
        ref[pl.ds(i * 16, 16)] = z
        return 0

    lax.fori_loop(0, n // 16, body, 0, unroll=8)


def _sc_deg_pack_body(src_hbm, dst_hbm, w_hbm, degp_hbm, pk_hbm, wq_hbm,
                      src_v, dst_v, w_v, pk_v, wq_v, acc_v):
    wid = _worker_id()
    _zero_f32(acc_v, NPAD)
    base = wid * EPW
    for ci in range(EPW // ACH):
        off = pl.multiple_of(base + ci * ACH, 8)
        pltpu.sync_copy(src_hbm.at[pl.ds(off, ACH)], src_v)
        pltpu.sync_copy(dst_hbm.at[pl.ds(off, ACH)], dst_v)
        pltpu.sync_copy(w_hbm.at[pl.ds(off, ACH)], w_v)

        def body(i, _):
            w0 = w_v[pl.ds(i * 32, 16)]
            w1 = w_v[pl.ds(i * 32 + 16, 16)]
            for sl, wv in ((pl.ds(i * 32, 16), w0),
                           (pl.ds(i * 32 + 16, 16), w1)):
                sv = src_v[sl]
                dv = dst_v[sl]
                plsc.addupdate_scatter(acc_v, [dv], wv)
                pk_v[sl] = jnp.bitwise_or(sv, lax.shift_left(dv, 16))
            u0 = lax.bitcast_convert_type(w0, jnp.int32) + 0x8000
            u1 = lax.bitcast_convert_type(w1, jnp.int32) + 0x8000
            wq_v[pl.ds(i * 16, 16)] = jnp.bitwise_or(
                lax.shift_right_logical(u0, 16), jnp.bitwise_and(u1, -65536))
            return 0

        lax.fori_loop(0, ACH // 32, body, 0, unroll=4)
        pltpu.sync_copy(pk_v, pk_hbm.at[pl.ds(off, ACH)])
        woff = pl.multiple_of(wid * (EPW // 2) + ci * (ACH // 2), 8)
        pltpu.sync_copy(wq_v, wq_hbm.at[pl.ds(woff, ACH // 2)])
    pltpu.sync_copy(acc_v, degp_hbm.at[wid])


def _sc_deg_pack(src, dst, w):
    return pl.kernel(
        _sc_deg_pack_body,
        out_type=(
            jax.ShapeDtypeStruct((NWORK, NPAD), jnp.float32),
            jax.ShapeDtypeStruct((EP,), jnp.int32),
            jax.ShapeDtypeStruct((EP // 2,), jnp.int32),
        ),
        mesh=plsc.VectorSubcoreMesh(core_axis_name="c", subcore_axis_name="s"),
        scratch_types=(
            pltpu.VMEM((ACH,), jnp.int32),
            pltpu.VMEM((ACH,), jnp.int32),
            pltpu.VMEM((ACH,), jnp.float32),
            pltpu.VMEM((ACH,), jnp.int32),
            pltpu.VMEM((ACH // 2,), jnp.int32),
            pltpu.VMEM((NPAD,), jnp.float32),
        ),
        compiler_params=pltpu.CompilerParams(needs_layout_passes=False),
    )(src, dst, w)


def _sc_msg_body(pk_hbm, wq_hbm, xsT_hbm, accT_hbm, pk_v, wq0_v, wq1_v, xs_v,
                 acc_v, sem0, sem1):
    wid = _worker_id()
    nch = EP // CCH
    sems = (sem0, sem1)
    wqs = (wq0_v, wq1_v)

    def start(ci, b):
        off = pl.multiple_of(ci * CCH, 8)
        woff = pl.multiple_of(ci * (CCH // 2), 8)
        pltpu.make_async_copy(pk_hbm.at[pl.ds(off, CCH)], pk_v.at[b],
                              sems[b]).start()
        pltpu.make_async_copy(wq_hbm.at[pl.ds(woff, CCH // 2)], wqs[b],
                              sems[b]).start()

    def wait(b):
        pltpu.make_async_copy(pk_hbm.at[pl.ds(0, CCH)], pk_v.at[b],
                              sems[b]).wait()
        pltpu.make_async_copy(wq_hbm.at[pl.ds(0, CCH // 2)], wqs[b],
                              sems[b]).wait()

    for fi in range(2):
        f = wid * 2 + fi
        start(0, 0)
        start(1, 1)
        pltpu.sync_copy(xsT_hbm.at[f], xs_v)
        _zero_f32(acc_v, NPAD)

        def outer(g, _):
            for b in range(2):
                ci = 2 * g + b
                wait(b)

                @plsc.parallel_loop(0, CCH // 2, step=16, unroll=4)
                def body(j):
                    word = wqs[b][pl.ds(j, 16)]
                    w0 = lax.bitcast_convert_type(
                        lax.shift_left(word, 16), jnp.float32)
                    w1 = lax.bitcast_convert_type(
                        jnp.bitwise_and(word, -65536), jnp.float32)
                    e = pl.multiple_of(j * 2, 32)
                    for k, wv in ((0, w0), (1, w1)):
                        pk = pk_v[b, pl.ds(e + k * 16, 16)]
                        sv = jnp.bitwise_and(pk, 0xFFFF)
                        dv = lax.shift_right_logical(pk, 16)
                        g16 = plsc.load_gather(xs_v, [sv])
                        plsc.addupdate_scatter(acc_v, [dv], g16 * wv)

                @pl.when(ci + 2 < nch)
                def _():
                    start(ci + 2, b)
            return 0

        lax.fori_loop(0, nch // 2, outer, 0)
        pltpu.sync_copy(acc_v, accT_hbm.at[f])


def _sc_msg(pk, wq, xsT):
    return pl.kernel(
        _sc_msg_body,
        out_type=jax.ShapeDtypeStruct((F, NPAD), jnp.float32),
        mesh=plsc.VectorSubcoreMesh(core_axis_name="c", subcore_axis_name="s"),
        scratch_types=(
            pltpu.VMEM((2, CCH), jnp.int32),
            pltpu.VMEM((CCH // 2,), jnp.int32),
            pltpu.VMEM((CCH // 2,), jnp.int32),
            pltpu.VMEM((NPAD,), jnp.float32),
            pltpu.VMEM((NPAD,), jnp.float32),
            pltpu.SemaphoreType.DMA,
            pltpu.SemaphoreType.DMA,
        ),
        compiler_params=pltpu.CompilerParams(needs_layout_passes=False),
    )(pk, wq, xsT)


def _tc_first_body(x_ref, degp_ref, w1_ref, xsT_ref, dis_ref):
    deg = jnp.sum(degp_ref[...], axis=0, keepdims=True) + 1.0
    dis = jnp.where(deg > 0, lax.rsqrt(deg), 0.0)
    xwT = lax.dot_general(w1_ref[...], x_ref[...], (((0,), (1,)), ((), ())))
    xsT_ref[...] = xwT * dis
    dis_ref[...] = dis


def _tc_first(x_p, degp, W1):
    cin = W1.shape[0]
    return pl.pallas_call(
        _tc_first_body,
        grid=(NBLK,),
        in_specs=[
            pl.BlockSpec((BLK, cin), lambda i: (i, 0)),
            pl.BlockSpec((NWORK, BLK), lambda i: (0, i)),
            pl.BlockSpec((cin, F), lambda i: (0, 0)),
        ],
        out_specs=[
            pl.BlockSpec((F, BLK), lambda i: (0, i)),
            pl.BlockSpec((1, BLK), lambda i: (0, i)),
        ],
        out_shape=[
            jax.ShapeDtypeStruct((F, NPAD), jnp.float32),
            jax.ShapeDtypeStruct((1, NPAD), jnp.float32),
        ],
    )(x_p, degp, W1)


def _tc_layer_body(accT_ref, xsT_ref, dis_ref, w_ref, bT_ref, out_ref):
    dis = dis_ref[...]
    hT = (accT_ref[...] + xsT_ref[...]) * dis + bT_ref[...]
    hT = jnp.maximum(hT, 0.0)
    xwT = lax.dot_general(w_ref[...], hT, (((0,), (0,)), ((), ())))
    out_ref[...] = xwT * dis


def _tc_layer(accT, xsT, dis, W, bT):
    return pl.pallas_call(
        _tc_layer_body,
        grid=(NBLK,),
        in_specs=[
            pl.BlockSpec((F, BLK), lambda i: (0, i)),
            pl.BlockSpec((F, BLK), lambda i: (0, i)),
            pl.BlockSpec((1, BLK), lambda i: (0, i)),
            pl.BlockSpec((F, F), lambda i: (0, 0)),
            pl.BlockSpec((F, 1), lambda i: (0, 0)),
        ],
        out_specs=pl.BlockSpec((F, BLK), lambda i: (0, i)),
        out_shape=jax.ShapeDtypeStruct((F, NPAD), jnp.float32),
    )(accT, xsT, dis, W, bT)


def _tc_pool_body(accT_ref, xsT_ref, dis_ref, bT_ref, batch_ref, w7_ref,
                  b7_ref, out_ref, sums_s, counts_s):
    i = pl.program_id(0)

    @pl.when(i == 0)
    def _():
        sums_s[...] = jnp.zeros_like(sums_s)
        counts_s[...] = jnp.zeros_like(counts_s)

    hT = (accT_ref[...] + xsT_ref[...]) * dis_ref[...] + bT_ref[...]
    gids = lax.broadcasted_iota(jnp.int32, (NG, BLK), 0)
    m = (batch_ref[...] == gids).astype(jnp.float32)
    sums_s[...] += lax.dot_general(m, hT, (((1,), (1,)), ((), ())))
    counts_s[...] += jnp.broadcast_to(
        jnp.sum(m, axis=1, keepdims=True), (NG, 128))

    @pl.when(i == pl.num_programs(0) - 1)
    def _():
        g = sums_s[...] / jnp.maximum(counts_s[:, 0:1], 1.0)
        out_ref[...] = jnp.dot(g, w7_ref[...]) + b7_ref[...]


def _tc_pool(accT, xsT, dis, bT, batch_p, W7, b7_2d):
    return pl.pallas_call(
        _tc_pool_body,
        grid=(NBLK,),
        in_specs=[
            pl.BlockSpec((F, BLK), lambda i: (0, i)),
            pl.BlockSpec((F, BLK), lambda i: (0, i)),
            pl.BlockSpec((1, BLK), lambda i: (0, i)),
            pl.BlockSpec((F, 1), lambda i: (0, 0)),
            pl.BlockSpec((1, BLK), lambda i: (0, i)),
            pl.BlockSpec((F, 2), lambda i: (0, 0)),
            pl.BlockSpec((1, 2), lambda i: (0, 0)),
        ],
        out_specs=pl.BlockSpec((NG, 2), lambda i: (0, 0)),
        out_shape=jax.ShapeDtypeStruct((NG, 2), jnp.float32),
        scratch_shapes=[
            pltpu.VMEM((NG, F), jnp.float32),
            pltpu.VMEM((NG, 128), jnp.float32),
        ],
    )(accT, xsT, dis, bT, batch_p, W7, b7_2d)


def kernel(x, edge_index, edge_attr, batch, W1, b1, W3, b3, W5, b5, W7, b7):
    N = x.shape[0]
    E = edge_index.shape[1]
    src = edge_index[0].astype(jnp.int32)
    dst = edge_index[1].astype(jnp.int32)
    pad_i = jnp.full((EP - E,), N, jnp.int32)
    src_p = jnp.concatenate([src, pad_i])
    dst_p = jnp.concatenate([dst, pad_i])
    w_p = jnp.concatenate([edge_attr.astype(jnp.float32),
                           jnp.zeros((EP - E,), jnp.float32)])
    x_p = jnp.pad(x.astype(jnp.float32), ((0, NPAD - N), (0, 0)))
    batch_p = jnp.concatenate(
        [batch.astype(jnp.int32), jnp.full((NPAD - N,), NG, jnp.int32)]
    ).reshape(1, NPAD)

    degp, pk, wq = _sc_deg_pack(src_p, dst_p, w_p)
    xs1T, dis = _tc_first(x_p, degp, W1)
    acc1T = _sc_msg(pk, wq, xs1T)
    xs2T = _tc_layer(acc1T, xs1T, dis, W3, b1.reshape(F, 1))
    acc2T = _sc_msg(pk, wq, xs2T)
    xs3T = _tc_layer(acc2T, xs2T, dis, W5, b3.reshape(F, 1))
    acc3T = _sc_msg(pk, wq, xs3T)
    return _tc_pool(acc3T, xs3T, dis, b5.reshape(F, 1), batch_p, W7,
                    b7.reshape(1, 2))
